# Initial kernel scaffold; baseline (speedup 1.0000x reference)
import jax
import jax.numpy as jnp
from jax.experimental import pallas as pl


def kernel(f_img, f_txt, W_fc, ln1_w, ln1_b, ln_img_w, ln_img_b, ln_txt_w, ln_txt_b):
    raise NotImplementedError("write your pallas kernel here")


# speedup vs baseline: 8.2987x; 8.2987x over previous
import functools

import jax
import jax.numpy as jnp
from jax.experimental import pallas as pl

BI = 4096
BT = 4096
DIM = 128
TOPK = 8
TAU = 0.07
SLW = 2.0
ALPHA = 1.0
EPS = 1e-06
LN_EPS = 1e-05

BLK = 256
GRID = BI // BLK
NEG_MASK = -1e9
NEG_INF = -1e30
BIG_I = 1 << 30

_HI = jax.lax.Precision.HIGHEST


def _l2n(x):
    n = jnp.sqrt(jnp.sum(x * x, axis=1, keepdims=True))
    return x / jnp.maximum(n, 1e-12)


def _sim_block(a_blk, b_full):
    fa = _l2n(a_blk)
    fb = _l2n(b_full)
    s = jax.lax.dot_general(fa, fb, (((1,), (1,)), ((), ())),
                            preferred_element_type=jnp.float32,
                            precision=jax.lax.Precision.DEFAULT)
    return s / TAU


def _top8_plain(S):
    col = jax.lax.broadcasted_iota(jnp.int32, S.shape, 1)
    Sw = S
    idxs = []
    amask = jnp.zeros(S.shape, dtype=jnp.bool_)
    for _ in range(TOPK):
        m = jnp.max(Sw, axis=1, keepdims=True)
        sel = Sw == m
        idx = jnp.min(jnp.where(sel, col, BIG_I), axis=1, keepdims=True)
        one = col == idx
        amask = amask | one
        Sw = jnp.where(one, NEG_INF, Sw)
        idxs.append(idx)
    return idxs, amask


def _top8_masked(S, mask):
    col = jax.lax.broadcasted_iota(jnp.int32, S.shape, 1)
    Sm = jnp.where(mask, S, NEG_MASK)
    idxs, vals = [], []
    for _ in range(TOPK):
        m = jnp.max(Sm, axis=1, keepdims=True)
        sel = Sm == m
        idx = jnp.min(jnp.where(sel, col, BIG_I), axis=1, keepdims=True)
        one = col == idx
        val = jnp.max(jnp.where(one, S, NEG_INF), axis=1, keepdims=True)
        Sm = jnp.where(one, NEG_INF, Sm)
        idxs.append(idx)
        vals.append(val)
    return idxs, vals


def _softmax8(vals):
    v = jnp.concatenate(vals, axis=1)
    m = jnp.max(v, axis=1, keepdims=True)
    e = jnp.exp(v - m)
    return e / jnp.sum(e, axis=1, keepdims=True)


def _cross_mask(idxT_ref, row_ids):
    m = None
    for j in range(TOPK):
        eq = idxT_ref[j:j + 1, :] == row_ids
        m = eq if m is None else (m | eq)
    return m


def _k1_body(a_ref, b_ref, idx0_ref):
    S = _sim_block(a_ref[...], b_ref[...])
    idxs, _ = _top8_plain(S)
    idx0_ref[...] = jnp.concatenate(idxs, axis=1)


def _k2_body(a_ref, b_ref, crossT_ref, idx0_ref, idx2_ref, w_ref, dv_ref):
    b = pl.program_id(0)
    S = _sim_block(a_ref[...], b_ref[...])
    idxs0, amask = _top8_plain(S)
    row_ids = b * BLK + jax.lax.broadcasted_iota(jnp.int32, (BLK, 1), 0)
    bmask = _cross_mask(crossT_ref, row_ids)
    idxs, vals = _top8_masked(S, amask & bmask)
    w = _softmax8(vals)
    idx0_ref[...] = jnp.concatenate(idxs0, axis=1)
    idx2_ref[...] = jnp.concatenate(idxs, axis=1)
    w_ref[...] = w
    col = jax.lax.broadcasted_iota(jnp.int32, (BLK, BT), 1)
    acc = jnp.zeros((BLK, BT), jnp.float32)
    for j in range(TOPK):
        acc = acc + jnp.where(col == idxs[j], w[:, j:j + 1], 0.0)
    part = jnp.sum(acc, axis=0, keepdims=True)

    @pl.when(b == 0)
    def _():
        dv_ref[...] = jnp.zeros_like(dv_ref)

    dv_ref[...] += part


def _k3_body(a_ref, b_ref, own_idx_ref, crossT_ref, idx2_ref, w_ref, dv_ref):
    b = pl.program_id(0)
    S = _sim_block(a_ref[...], b_ref[...])
    col = jax.lax.broadcasted_iota(jnp.int32, (BLK, BT), 1)
    own = own_idx_ref[...]
    amask = None
    for j in range(TOPK):
        eq = col == own[:, j:j + 1]
        amask = eq if amask is None else (amask | eq)
    row_ids = b * BLK + jax.lax.broadcasted_iota(jnp.int32, (BLK, 1), 0)
    bmask = _cross_mask(crossT_ref, row_ids)
    idxs, vals = _top8_masked(S, amask & bmask)
    w = _softmax8(vals)
    idx2_ref[...] = jnp.concatenate(idxs, axis=1)
    w_ref[...] = w
    acc = jnp.zeros((BLK, BT), jnp.float32)
    for j in range(TOPK):
        acc = acc + jnp.where(col == idxs[j], w[:, j:j + 1], 0.0)
    part = jnp.sum(acc, axis=0, keepdims=True)

    @pl.when(b == 0)
    def _():
        dv_ref[...] = jnp.zeros_like(dv_ref)

    dv_ref[...] += part


def _xe_body(idx_ref, w_ref, xo_ref, xs_ref, deinv_ref, out_ref):
    col = jax.lax.broadcasted_iota(jnp.int32, (BLK, BT), 1)
    idx = idx_ref[...]
    w = w_ref[...]
    G = jnp.zeros((BLK, BT), jnp.float32)
    for j in range(TOPK):
        G = G + jnp.where(col == idx[:, j:j + 1], w[:, j:j + 1], 0.0)
    agg = jax.lax.dot_general(G, xo_ref[...], (((1,), (0,)), ((), ())),
                              preferred_element_type=jnp.float32,
                              precision=_HI)
    out_ref[...] = (SLW * xs_ref[...] + agg) * deinv_ref[...]


def _x1_body(idxT_ref, wT_ref, xe_o_ref, xe_s_ref, dvis_ref, out_ref):
    b = pl.program_id(0)
    row_ids = b * BLK + jax.lax.broadcasted_iota(jnp.int32, (BLK, 1), 0)
    GT = jnp.zeros((BLK, BT), jnp.float32)
    for j in range(TOPK):
        eq = idxT_ref[j:j + 1, :] == row_ids
        GT = GT + jnp.where(eq, wT_ref[j:j + 1, :], 0.0)
    agg = jax.lax.dot_general(GT, xe_o_ref[...], (((1,), (0,)), ((), ())),
                              preferred_element_type=jnp.float32,
                              precision=_HI)
    out_ref[...] = (SLW * xe_s_ref[...] + agg) * dvis_ref[...]


def _final_body(x_ref, x1_ref, wfc_ref, ln1w_ref, ln1b_ref, lnsw_ref,
                lnsb_ref, out_ref):
    y = jax.lax.dot_general(x1_ref[...], wfc_ref[...],
                            (((1,), (1,)), ((), ())),
                            preferred_element_type=jnp.float32,
                            precision=_HI)
    g = 0.5 * y * (1.0 + jax.lax.erf(y / jnp.sqrt(2.0).astype(jnp.float32)))
    r = x_ref[...] + ALPHA * g

    def ln(v, wr, br):
        mu = jnp.mean(v, axis=1, keepdims=True)
        var = jnp.mean((v - mu) ** 2, axis=1, keepdims=True)
        return (v - mu) / jnp.sqrt(var + LN_EPS) * wr[...] + br[...]

    out_ref[...] = ln(ln(r, ln1w_ref, ln1b_ref), lnsw_ref, lnsb_ref)


def _spec_blk(w=DIM):
    return pl.BlockSpec((BLK, w), lambda b: (b, 0))


def _spec_full(h, w):
    return pl.BlockSpec((h, w), lambda b: (0, 0))


def kernel(f_img, f_txt, W_fc, ln1_w, ln1_b, ln_img_w, ln_img_b, ln_txt_w,
           ln_txt_b):
    f32 = jnp.float32
    i32 = jnp.int32

    idx_t0 = pl.pallas_call(
        _k1_body,
        grid=(GRID,),
        in_specs=[_spec_blk(), _spec_full(BT, DIM)],
        out_specs=_spec_blk(TOPK),
        out_shape=jax.ShapeDtypeStruct((BI, TOPK), i32),
    )(f_img, f_txt)

    idx_t0T = idx_t0.T
    idx_i0, idx_i2, w_i, dv_img_s = pl.pallas_call(
        _k2_body,
        grid=(GRID,),
        in_specs=[_spec_blk(), _spec_full(BI, DIM), _spec_full(TOPK, BI)],
        out_specs=[_spec_blk(TOPK), _spec_blk(TOPK), _spec_blk(TOPK),
                   _spec_full(1, BI)],
        out_shape=[jax.ShapeDtypeStruct((BT, TOPK), i32),
                   jax.ShapeDtypeStruct((BT, TOPK), i32),
                   jax.ShapeDtypeStruct((BT, TOPK), f32),
                   jax.ShapeDtypeStruct((1, BI), f32)],
    )(f_txt, f_img, idx_t0T)

    idx_i0T = idx_i0.T
    idx_t2, w_t, dv_txt_s = pl.pallas_call(
        _k3_body,
        grid=(GRID,),
        in_specs=[_spec_blk(), _spec_full(BT, DIM), _spec_blk(TOPK),
                  _spec_full(TOPK, BT)],
        out_specs=[_spec_blk(TOPK), _spec_blk(TOPK), _spec_full(1, BT)],
        out_shape=[jax.ShapeDtypeStruct((BI, TOPK), i32),
                   jax.ShapeDtypeStruct((BI, TOPK), f32),
                   jax.ShapeDtypeStruct((1, BT), f32)],
    )(f_img, f_txt, idx_t0, idx_i0T)

    dv_img = SLW + dv_img_s[0]
    dv_txt = SLW + dv_txt_s[0]
    de_img = SLW + jnp.sum(w_t, axis=1)
    de_txt = SLW + jnp.sum(w_i, axis=1)
    dvis_img = (1.0 / jnp.sqrt(dv_img + EPS))[:, None]
    dvis_txt = (1.0 / jnp.sqrt(dv_txt + EPS))[:, None]
    deinv_img = (1.0 / (de_img + EPS))[:, None]
    deinv_txt = (1.0 / (de_txt + EPS))[:, None]
    x0_img = f_img * dvis_img
    x0_txt = f_txt * dvis_txt

    def xe_call(idx, w, x_other, x_self, deinv):
        return pl.pallas_call(
            _xe_body,
            grid=(GRID,),
            in_specs=[_spec_blk(TOPK), _spec_blk(TOPK), _spec_full(BT, DIM),
                      _spec_blk(), _spec_blk(1)],
            out_specs=_spec_blk(),
            out_shape=jax.ShapeDtypeStruct((BI, DIM), f32),
        )(idx, w, x_other, x_self, deinv)

    xe_img = xe_call(idx_t2, w_t, x0_txt, x0_img, deinv_img)
    xe_txt = xe_call(idx_i2, w_i, x0_img, x0_txt, deinv_txt)

    def x1_call(idxT, wT, xe_other, xe_self, dvis):
        return pl.pallas_call(
            _x1_body,
            grid=(GRID,),
            in_specs=[_spec_full(TOPK, BI), _spec_full(TOPK, BI),
                      _spec_full(BT, DIM), _spec_blk(), _spec_blk(1)],
            out_specs=_spec_blk(),
            out_shape=jax.ShapeDtypeStruct((BI, DIM), f32),
        )(idxT, wT, xe_other, xe_self, dvis)

    x1_txt = x1_call(idx_t2.T, w_t.T, xe_img, xe_txt, dvis_txt)
    x1_img = x1_call(idx_i2.T, w_i.T, xe_txt, xe_img, dvis_img)

    def final_call(x, x1, lnsw, lnsb):
        return pl.pallas_call(
            _final_body,
            grid=(GRID,),
            in_specs=[_spec_blk(), _spec_blk(), _spec_full(DIM, DIM),
                      _spec_full(1, DIM), _spec_full(1, DIM),
                      _spec_full(1, DIM), _spec_full(1, DIM)],
            out_specs=_spec_blk(),
            out_shape=jax.ShapeDtypeStruct((BI, DIM), f32),
        )(x, x1, W_fc, ln1_w[None, :], ln1_b[None, :], lnsw[None, :],
          lnsb[None, :])

    out_img = final_call(f_img, x1_img, ln_img_w, ln_img_b)
    out_txt = final_call(f_txt, x1_txt, ln_txt_w, ln_txt_b)
    return (out_img, out_txt)
